# baseline (speedup 1.0000x reference)
import functools

import jax
import jax.numpy as jnp
from jax import lax
from jax.experimental import pallas as pl
from jax.experimental.pallas import tpu as pltpu
from jax.experimental.pallas import tpu_sc as plsc

VOCAB = 50257
EMBED_DIM = 768
EPS = 1e-12

_NC = 2
_NS = 16
_NW = _NC * _NS

_CHUNK = 32
_NSLOT = 3
_PACKD = EMBED_DIM // 2


def _norm_pack_body(x_ref, o_ref):
    x = x_ref[...]
    ssq = jnp.sum(x * x, axis=1, keepdims=True)
    xn = x / jnp.maximum(jnp.sqrt(ssq), EPS)
    xb = xn.astype(jnp.bfloat16)
    u = lax.bitcast_convert_type(xb, jnp.uint16).astype(jnp.int32)
    u3 = u.reshape(u.shape[0], EMBED_DIM // 32, 32)
    w = (u3[:, :, 16:] << 16) | u3[:, :, :16]
    o_ref[...] = w.reshape(u.shape[0], _PACKD)


def _normalize_pack_table(table):
    rows, d = table.shape
    br = 1024
    return pl.pallas_call(
        _norm_pack_body,
        grid=(pl.cdiv(rows, br),),
        in_specs=[pl.BlockSpec((br, d), lambda i: (i, 0))],
        out_specs=pl.BlockSpec((br, _PACKD), lambda i: (i, 0)),
        out_shape=jax.ShapeDtypeStruct((rows, _PACKD), jnp.int32),
    )(table)


def _gather_body(n_chunks, ids_hbm, tab_hbm, out_hbm,
                 idx0, idx1, idx2, pk0, pk1, pk2, fb0, fb1, fb2,
                 gsem0, gsem1, gsem2, osem0, osem1, osem2):
    wid = lax.axis_index("s") * _NC + lax.axis_index("c")
    per_w = n_chunks * _CHUNK
    base = wid * per_w

    idx_l = (idx0, idx1, idx2)
    pk_l = (pk0, pk1, pk2)
    fb_l = (fb0, fb1, fb2)
    gsem_l = (gsem0, gsem1, gsem2)
    osem_l = (osem0, osem1, osem2)

    def gather_cp(s):
        return pltpu.make_async_copy(tab_hbm.at[idx_l[s]], pk_l[s], gsem_l[s])

    def issue_gather(g, s):
        pltpu.sync_copy(ids_hbm.at[pl.ds(base + g * _CHUNK, _CHUNK)], idx_l[s])
        gather_cp(s).start()

    def write_cp(g, s):
        out_view = out_hbm.at[pl.ds(base + g * _CHUNK, _CHUNK)]
        return pltpu.make_async_copy(fb_l[s], out_view, osem_l[s])

    def depack(s):
        pk = pk_l[s]
        fb = fb_l[s]
        mask = jnp.int32(-65536)

        def row(r, carry):
            for u in range(_PACKD // 16):
                w = pk[r, pl.ds(u * 16, 16)]
                fb[r, pl.ds(u * 32, 16)] = w << 16
                fb[r, pl.ds(u * 32 + 16, 16)] = w & mask
            return carry

        lax.fori_loop(0, _CHUNK, row, None)

    def run_chunk(g, s, issue_next, wait_old_write):
        gather_cp(s).wait()
        if issue_next is not None:
            issue_next()
        if wait_old_write:
            write_cp(g, s).wait()
        depack(s)
        write_cp(g, s).start()

    issue_gather(0, 0)
    issue_gather(1, 1)

    n_main = (n_chunks - 2) // _NSLOT * _NSLOT

    def step(i, _):
        for u in range(_NSLOT):
            g = i * _NSLOT + u

            def nxt(g=g, u=u):
                @pl.when(g + 2 < n_chunks)
                def _():
                    issue_gather(g + 2, (u + 2) % _NSLOT)

            @pl.when(i > 0)
            def _(g=g, u=u, nxt=nxt):
                run_chunk(g, u, nxt, wait_old_write=True)

            @pl.when(i == 0)
            def _(g=g, u=u, nxt=nxt):
                run_chunk(g, u, nxt, wait_old_write=False)
        return _

    lax.fori_loop(0, n_main // _NSLOT, step, None)

    for g in range(n_main, n_chunks):
        s = g % _NSLOT
        if g + 2 < n_chunks:
            run_chunk(g, s, lambda g=g, s=s: issue_gather(g + 2, (g + 2) % _NSLOT),
                      wait_old_write=True)
        else:
            run_chunk(g, s, None, wait_old_write=True)

    for g in range(n_chunks - _NSLOT, n_chunks):
        write_cp(g, g % _NSLOT).wait()


def _gather_rows(table_pk, flat_ids):
    b_tot = flat_ids.shape[0]
    per_w = b_tot // _NW
    n_chunks = per_w // _CHUNK

    mesh = plsc.VectorSubcoreMesh(
        core_axis_name="c", subcore_axis_name="s",
        num_cores=_NC, num_subcores=_NS)

    grab = pl.kernel(
        functools.partial(_gather_body, n_chunks),
        out_type=jax.ShapeDtypeStruct((b_tot, EMBED_DIM), jnp.int32),
        mesh=mesh,
        compiler_params=pltpu.CompilerParams(use_tc_tiling_on_sc=False),
        scratch_types=(
            [pltpu.VMEM((_CHUNK,), jnp.int32)] * _NSLOT
            + [pltpu.VMEM((_CHUNK, _PACKD), jnp.int32)] * _NSLOT
            + [pltpu.VMEM((_CHUNK, EMBED_DIM), jnp.int32)] * _NSLOT
            + [pltpu.SemaphoreType.DMA] * (2 * _NSLOT)
        ),
    )
    return grab(flat_ids, table_pk)


def kernel(token_ids, table):
    bsz, seq = token_ids.shape
    table_pk = _normalize_pack_table(table)
    flat_ids = token_ids.reshape(-1).astype(jnp.int32)
    out = _gather_rows(table_pk, flat_ids)
    out = lax.bitcast_convert_type(out, jnp.float32)
    return out.reshape(bsz, seq, EMBED_DIM)


# speedup vs baseline: 1.6159x; 1.6159x over previous
import functools

import jax
import jax.numpy as jnp
from jax import lax
from jax.experimental import pallas as pl
from jax.experimental.pallas import tpu as pltpu
from jax.experimental.pallas import tpu_sc as plsc

VOCAB = 50257
EMBED_DIM = 768
EPS = 1e-12

_NC = 2
_NS = 16
_NW = _NC * _NS

_CHUNK = 16
_NSLOT = 3
_PACKD = EMBED_DIM // 2


def _norm_pack_body(x_ref, o_ref):
    x = x_ref[...]
    ssq = jnp.sum(x * x, axis=1, keepdims=True)
    xn = x / jnp.maximum(jnp.sqrt(ssq), EPS)
    xb = xn.astype(jnp.bfloat16)
    u = lax.bitcast_convert_type(xb, jnp.uint16).astype(jnp.int32)
    u3 = u.reshape(u.shape[0], EMBED_DIM // 32, 32)
    w = (u3[:, :, 16:] << 16) | u3[:, :, :16]
    o_ref[...] = w.reshape(u.shape[0], _PACKD)


def _normalize_pack_table(table):
    rows, d = table.shape
    br = 1024
    return pl.pallas_call(
        _norm_pack_body,
        grid=(pl.cdiv(rows, br),),
        in_specs=[pl.BlockSpec((br, d), lambda i: (i, 0))],
        out_specs=pl.BlockSpec((br, _PACKD), lambda i: (i, 0)),
        out_shape=jax.ShapeDtypeStruct((rows, _PACKD), jnp.int32),
    )(table)


def _gather_body(n_chunks, ids_hbm, tab_hbm, out_hbm,
                 idx0, idx1, idx2, pk0, pk1, pk2, fb0, fb1, fb2,
                 gsem0, gsem1, gsem2, osem0, osem1, osem2):
    wid = lax.axis_index("s") * _NC + lax.axis_index("c")
    per_w = n_chunks * _CHUNK
    base = wid * per_w

    idx_l = (idx0, idx1, idx2)
    pk_l = (pk0, pk1, pk2)
    fb_l = (fb0, fb1, fb2)
    gsem_l = (gsem0, gsem1, gsem2)
    osem_l = (osem0, osem1, osem2)

    def gather_cp(s):
        return pltpu.make_async_copy(tab_hbm.at[idx_l[s]], pk_l[s], gsem_l[s])

    def issue_gather(g, s):
        pltpu.sync_copy(ids_hbm.at[pl.ds(base + g * _CHUNK, _CHUNK)], idx_l[s])
        gather_cp(s).start()

    def write_cp(g, s):
        out_view = out_hbm.at[pl.ds(base + g * _CHUNK, _CHUNK)]
        return pltpu.make_async_copy(fb_l[s], out_view, osem_l[s])

    def depack(s):
        pk = pk_l[s]
        fb = fb_l[s]
        mask = jnp.int32(-65536)
        for r in range(_CHUNK):
            for u in range(_PACKD // 16):
                w = pk[r, pl.ds(u * 16, 16)]
                fb[r, pl.ds(u * 32, 16)] = w << 16
                fb[r, pl.ds(u * 32 + 16, 16)] = w & mask

    def run_chunk(g, s, guard):
        gather_cp(s).wait()
        if guard:
            @pl.when(g + 2 < n_chunks)
            def _():
                issue_gather(g + 2, (s + 2) % _NSLOT)

            @pl.when(g >= _NSLOT)
            def _():
                write_cp(g, s).wait()
        else:
            if g + 2 < n_chunks:
                issue_gather(g + 2, (s + 2) % _NSLOT)
            if g >= _NSLOT:
                write_cp(g, s).wait()
        depack(s)
        write_cp(g, s).start()

    issue_gather(0, 0)
    issue_gather(1, 1)

    n_main = n_chunks // _NSLOT * _NSLOT

    def step(i, _):
        for u in range(_NSLOT):
            run_chunk(i * _NSLOT + u, u, guard=True)
        return _

    lax.fori_loop(0, n_main // _NSLOT, step, None)

    for g in range(n_main, n_chunks):
        run_chunk(g, g % _NSLOT, guard=False)

    for g in range(n_chunks - _NSLOT, n_chunks):
        write_cp(g, g % _NSLOT).wait()


def _gather_rows(table_pk, flat_ids):
    b_tot = flat_ids.shape[0]
    per_w = b_tot // _NW
    n_chunks = per_w // _CHUNK

    mesh = plsc.VectorSubcoreMesh(
        core_axis_name="c", subcore_axis_name="s",
        num_cores=_NC, num_subcores=_NS)

    grab = pl.kernel(
        functools.partial(_gather_body, n_chunks),
        out_type=jax.ShapeDtypeStruct((b_tot, EMBED_DIM), jnp.int32),
        mesh=mesh,
        scratch_types=(
            [pltpu.VMEM((_CHUNK,), jnp.int32)] * _NSLOT
            + [pltpu.VMEM((_CHUNK, _PACKD), jnp.int32)] * _NSLOT
            + [pltpu.VMEM((_CHUNK, EMBED_DIM), jnp.int32)] * _NSLOT
            + [pltpu.SemaphoreType.DMA] * (2 * _NSLOT)
        ),
    )
    return grab(flat_ids, table_pk)


def kernel(token_ids, table):
    bsz, seq = token_ids.shape
    table_pk = _normalize_pack_table(table)
    flat_ids = token_ids.reshape(-1).astype(jnp.int32)
    out = _gather_rows(table_pk, flat_ids)
    out = lax.bitcast_convert_type(out, jnp.float32)
    return out.reshape(bsz, seq, EMBED_DIM)


# speedup vs baseline: 1.9213x; 1.1889x over previous
import functools

import jax
import jax.numpy as jnp
from jax import lax
from jax.experimental import pallas as pl
from jax.experimental.pallas import tpu as pltpu
from jax.experimental.pallas import tpu_sc as plsc

VOCAB = 50257
EMBED_DIM = 768
EPS = 1e-12

_NC = 2
_NS = 16
_NW = _NC * _NS

_CHUNK = 32
_NSLOT = 2
_PACKD = EMBED_DIM // 2


def _norm_pack_body(x_ref, o_ref):
    x = x_ref[...]
    ssq = jnp.sum(x * x, axis=1, keepdims=True)
    xn = x / jnp.maximum(jnp.sqrt(ssq), EPS)
    xb = xn.astype(jnp.bfloat16)
    u = lax.bitcast_convert_type(xb, jnp.uint16).astype(jnp.int32)
    u3 = u.reshape(u.shape[0], EMBED_DIM // 32, 32)
    w = (u3[:, :, 16:] << 16) | u3[:, :, :16]
    o_ref[...] = w.reshape(u.shape[0], _PACKD)


def _normalize_pack_table(table):
    rows, d = table.shape
    br = 1024
    return pl.pallas_call(
        _norm_pack_body,
        grid=(pl.cdiv(rows, br),),
        in_specs=[pl.BlockSpec((br, d), lambda i: (i, 0))],
        out_specs=pl.BlockSpec((br, _PACKD), lambda i: (i, 0)),
        out_shape=jax.ShapeDtypeStruct((rows, _PACKD), jnp.int32),
    )(table)


def _gather_body(n_chunks, ids_hbm, tab_hbm, out_hbm,
                 idx_all, pk0, pk1, fb0, fb1,
                 gsem0, gsem1, osem0, osem1):
    wid = lax.axis_index("s") * _NC + lax.axis_index("c")
    per_w = n_chunks * _CHUNK
    base = wid * per_w

    pk_l = (pk0, pk1)
    fb_l = (fb0, fb1)
    gsem_l = (gsem0, gsem1)
    osem_l = (osem0, osem1)

    pltpu.sync_copy(ids_hbm.at[pl.ds(base, per_w)], idx_all)

    def gather_cp(g, s):
        idx_view = idx_all.at[pl.ds(g * _CHUNK, _CHUNK)]
        return pltpu.make_async_copy(tab_hbm.at[idx_view], pk_l[s], gsem_l[s])

    def write_cp(g, s):
        out_view = out_hbm.at[pl.ds(base + g * _CHUNK, _CHUNK)]
        return pltpu.make_async_copy(fb_l[s], out_view, osem_l[s])

    def depack(s):
        pk = pk_l[s]
        fb = fb_l[s]
        mask = jnp.int32(-65536)
        for r in range(_CHUNK):
            for u in range(_PACKD // 16):
                w = pk[r, pl.ds(u * 16, 16)]
                fb[r, pl.ds(u * 32, 16)] = w << 16
                fb[r, pl.ds(u * 32 + 16, 16)] = w & mask

    def run_chunk(g, s, guard):
        gather_cp(g, s).wait()
        if guard:
            @pl.when(g >= _NSLOT)
            def _():
                write_cp(g, s).wait()
        elif g >= _NSLOT:
            write_cp(g, s).wait()
        depack(s)
        write_cp(g, s).start()
        if guard:
            @pl.when(g + _NSLOT < n_chunks)
            def _():
                gather_cp(g + _NSLOT, s).start()
        elif g + _NSLOT < n_chunks:
            gather_cp(g + _NSLOT, s).start()

    gather_cp(0, 0).start()
    gather_cp(1, 1).start()

    n_main = n_chunks // _NSLOT * _NSLOT

    def step(i, _):
        for u in range(_NSLOT):
            run_chunk(i * _NSLOT + u, u, guard=True)
        return _

    lax.fori_loop(0, n_main // _NSLOT, step, None)

    for g in range(n_main, n_chunks):
        run_chunk(g, g % _NSLOT, guard=False)

    for g in range(n_chunks - _NSLOT, n_chunks):
        write_cp(g, g % _NSLOT).wait()


def _gather_rows(table_pk, flat_ids):
    b_tot = flat_ids.shape[0]
    per_w = b_tot // _NW
    n_chunks = per_w // _CHUNK

    mesh = plsc.VectorSubcoreMesh(
        core_axis_name="c", subcore_axis_name="s",
        num_cores=_NC, num_subcores=_NS)

    grab = pl.kernel(
        functools.partial(_gather_body, n_chunks),
        out_type=jax.ShapeDtypeStruct((b_tot, EMBED_DIM), jnp.int32),
        mesh=mesh,
        scratch_types=(
            [pltpu.VMEM((per_w,), jnp.int32)]
            + [pltpu.VMEM((_CHUNK, _PACKD), jnp.int32)] * _NSLOT
            + [pltpu.VMEM((_CHUNK, EMBED_DIM), jnp.int32)] * _NSLOT
            + [pltpu.SemaphoreType.DMA] * (2 * _NSLOT)
        ),
    )
    return grab(flat_ids, table_pk)


def kernel(token_ids, table):
    bsz, seq = token_ids.shape
    table_pk = _normalize_pack_table(table)
    flat_ids = token_ids.reshape(-1).astype(jnp.int32)
    out = _gather_rows(table_pk, flat_ids)
    out = lax.bitcast_convert_type(out, jnp.float32)
    return out.reshape(bsz, seq, EMBED_DIM)


# speedup vs baseline: 4.2384x; 2.2060x over previous
import functools

import jax
import jax.numpy as jnp
from jax import lax
from jax.experimental import pallas as pl
from jax.experimental.pallas import tpu as pltpu
from jax.experimental.pallas import tpu_sc as plsc

VOCAB = 50257
EMBED_DIM = 768
EPS = 1e-12

_NC = 2
_NS = 16
_NW = _NC * _NS

_CHUNK = 32
_NBUF = 4


def _normalize_body(x_ref, o_ref):
    x = x_ref[...]
    ssq = jnp.sum(x * x, axis=1, keepdims=True)
    o_ref[...] = x / jnp.maximum(jnp.sqrt(ssq), EPS)


def _normalize_table(table):
    rows, d = table.shape
    br = 1024
    return pl.pallas_call(
        _normalize_body,
        grid=(pl.cdiv(rows, br),),
        in_specs=[pl.BlockSpec((br, d), lambda i: (i, 0))],
        out_specs=pl.BlockSpec((br, d), lambda i: (i, 0)),
        out_shape=jax.ShapeDtypeStruct((rows, d), table.dtype),
    )(table)


def _gather_body(n_chunks, ids_hbm, tab_hbm, out_hbm, idx_all,
                 rows0, rows1, rows2, rows3,
                 gsem0, gsem1, gsem2, gsem3,
                 osem0, osem1, osem2, osem3):
    wid = lax.axis_index("s") * _NC + lax.axis_index("c")
    per_w = n_chunks * _CHUNK
    base = wid * per_w

    rows_l = (rows0, rows1, rows2, rows3)
    gsem_l = (gsem0, gsem1, gsem2, gsem3)
    osem_l = (osem0, osem1, osem2, osem3)

    pltpu.sync_copy(ids_hbm.at[pl.ds(base, per_w)], idx_all)

    def gather_cp(g, b):
        idx_view = idx_all.at[pl.ds(g * _CHUNK, _CHUNK)]
        return pltpu.make_async_copy(tab_hbm.at[idx_view], rows_l[b], gsem_l[b])

    def write_cp(g, b):
        out_view = out_hbm.at[pl.ds(base + g * _CHUNK, _CHUNK)]
        return pltpu.make_async_copy(rows_l[b], out_view, osem_l[b])

    for b in range(_NBUF - 1):
        gather_cp(b, b).start()

    def step(i, _):
        for b in range(_NBUF):
            g = i * _NBUF + b
            pb = (b - 1) % _NBUF
            gather_cp(g, b).wait()
            write_cp(g, b).start()

            @pl.when(g >= 1)
            def _():
                write_cp(g - 1, pb).wait()

            nxt = g + _NBUF - 1

            @pl.when(nxt < n_chunks)
            def _():
                gather_cp(nxt, pb).start()

        return _

    lax.fori_loop(0, n_chunks // _NBUF, step, None)
    write_cp(n_chunks - 1, (n_chunks - 1) % _NBUF).wait()


def _gather_rows(table_n, flat_ids):
    b_tot = flat_ids.shape[0]
    d = table_n.shape[1]
    per_w = b_tot // _NW
    n_chunks = per_w // _CHUNK

    mesh = plsc.VectorSubcoreMesh(
        core_axis_name="c", subcore_axis_name="s",
        num_cores=_NC, num_subcores=_NS)

    grab = pl.kernel(
        functools.partial(_gather_body, n_chunks),
        out_type=jax.ShapeDtypeStruct((b_tot, d), jnp.float32),
        mesh=mesh,
        scratch_types=(
            [pltpu.VMEM((per_w,), jnp.int32)]
            + [pltpu.VMEM((_CHUNK, d), jnp.float32)] * _NBUF
            + [pltpu.SemaphoreType.DMA] * (2 * _NBUF)
        ),
    )
    return grab(flat_ids, table_n)


def kernel(token_ids, table):
    bsz, seq = token_ids.shape
    table_n = _normalize_table(table)
    flat_ids = token_ids.reshape(-1).astype(jnp.int32)
    out = _gather_rows(table_n, flat_ids)
    return out.reshape(bsz, seq, EMBED_DIM)
